# Initial kernel scaffold; baseline (speedup 1.0000x reference)
import jax
import jax.numpy as jnp
from jax.experimental import pallas as pl


def kernel(beta, gamma, latent_zi, latent_zj, sample_i_idx, sample_j_idx, sparse_i_sample, sparse_j_sample):
    raise NotImplementedError("write your pallas kernel here")


# speedup vs baseline: 56.0923x; 56.0923x over previous
import functools

import jax
import jax.numpy as jnp
from jax import lax
from jax.experimental import pallas as pl
from jax.experimental.pallas import tpu as pltpu
from jax.experimental.pallas import tpu_sc as plsc

NC = 2
NS = 16
NW = NC * NS
LANES = 16

D = 8
AUG = 16
SUB = 128
NSUB = 8
CHUNK = SUB * NSUB

S_BLK = 256


def _ceil_to(x, m):
    return (x + m - 1) // m * m


def _vsqrt(x):
    x = jnp.maximum(x, 1e-30)
    i = plsc.bitcast(x, jnp.int32)
    i = jnp.int32(0x1FBD1DF5) + lax.shift_right_logical(i, 1)
    y = plsc.bitcast(i, jnp.float32)
    y = 0.5 * (y + x / y)
    y = 0.5 * (y + x / y)
    y = 0.5 * (y + x / y)
    return y


def _wid():
    return lax.axis_index("s") * NC + lax.axis_index("c")


def _link_kernel_body(n_chunks, e_valid, aug_i, aug_j, idx_i, idx_j, out,
                      idx_iv, idx_jv, rows_iv, rows_jv, acc_v, sem_i, sem_j):
    wid = _wid()
    row_base = wid * (n_chunks * NSUB)
    iota = lax.iota(jnp.int32, LANES)

    def chunk_body(c, acc):
        rb = row_base + c * NSUB
        pltpu.sync_copy(idx_i.at[pl.ds(rb, NSUB)], idx_iv)
        pltpu.sync_copy(idx_j.at[pl.ds(rb, NSUB)], idx_jv)
        copies = []
        for k in range(NSUB):
            copies.append(pltpu.async_copy(
                aug_i.at[idx_iv.at[k]], rows_iv.at[pl.ds(k * SUB, SUB)], sem_i))
            copies.append(pltpu.async_copy(
                aug_j.at[idx_jv.at[k]], rows_jv.at[pl.ds(k * SUB, SUB)], sem_j))
        for cp in copies:
            cp.wait()

        edge0 = (row_base + c * NSUB) * SUB

        def group_body(g, acc):
            r = g * LANES + iota
            d2 = jnp.zeros((LANES,), jnp.float32)
            for d in range(D):
                col = jnp.full((LANES,), d, jnp.int32)
                a = plsc.load_gather(rows_iv, [r, col])
                b = plsc.load_gather(rows_jv, [r, col])
                diff = a - b + 1e-6
                d2 = d2 + diff * diff
            colb = jnp.full((LANES,), D, jnp.int32)
            beta_v = plsc.load_gather(rows_iv, [r, colb])
            gamma_v = plsc.load_gather(rows_jv, [r, colb])
            lam = beta_v + gamma_v - _vsqrt(d2)
            valid = (edge0 + g * LANES + iota) < e_valid
            return acc + jnp.where(valid, lam, 0.0)

        return lax.fori_loop(0, CHUNK // LANES, group_body, acc)

    acc = lax.fori_loop(0, n_chunks, chunk_body, jnp.zeros((LANES,), jnp.float32))
    acc_v[...] = acc
    pltpu.sync_copy(acc_v, out.at[wid])


def _sample_kernel_body(s_per_tile, aug_i, aug_j, idx_i, idx_j, out_i, out_j,
                        idx_v, rows_v, sem):
    wid = _wid()
    base = wid * s_per_tile
    pltpu.sync_copy(idx_i.at[pl.ds(base, s_per_tile)], idx_v)
    pltpu.async_copy(aug_i.at[idx_v], rows_v, sem).wait()
    pltpu.sync_copy(rows_v, out_i.at[pl.ds(base, s_per_tile)])
    pltpu.sync_copy(idx_j.at[pl.ds(base, s_per_tile)], idx_v)
    pltpu.async_copy(aug_j.at[idx_v], rows_v, sem).wait()
    pltpu.sync_copy(rows_v, out_j.at[pl.ds(base, s_per_tile)])


def _dense_body(s_i, s_j, a_ref, b_ref, o_ref):
    i = pl.program_id(0)
    a = a_ref[...]
    b = b_ref[...]
    az = a[:, :D] + 1e-6
    bz = b[:, :D]
    a2 = jnp.sum(az * az, axis=1, keepdims=True)
    b2 = jnp.sum(bz * bz, axis=1)[None, :]
    cross = lax.dot_general(az, bz, (((1,), (1,)), ((), ())),
                            preferred_element_type=jnp.float32)
    d2 = jnp.maximum(a2 + b2 - 2.0 * cross, 0.0)
    lam = a[:, D][:, None] + b[:, D][None, :] - jnp.sqrt(d2)
    n_pad = b.shape[0]
    rows = i * S_BLK + lax.broadcasted_iota(jnp.int32, (S_BLK, n_pad), 0)
    cols = lax.broadcasted_iota(jnp.int32, (S_BLK, n_pad), 1)
    val = jnp.sum(jnp.where((rows < s_i) & (cols < s_j), jnp.exp(lam), 0.0))

    @pl.when(i == 0)
    def _():
        o_ref[...] = jnp.zeros((1, 1), jnp.float32)

    o_ref[...] = o_ref[...] + val


def kernel(beta, gamma, latent_zi, latent_zj, sample_i_idx, sample_j_idx,
           sparse_i_sample, sparse_j_sample):
    n_i, d = latent_zi.shape
    n_j, _ = latent_zj.shape
    s_i = sample_i_idx.shape[0]
    s_j = sample_j_idx.shape[0]
    e = sparse_i_sample.shape[0]
    f32 = jnp.float32

    aug_i = jnp.concatenate(
        [latent_zi, beta[:, None], jnp.zeros((n_i, AUG - d - 1), f32)], axis=1)
    aug_j = jnp.concatenate(
        [latent_zj, gamma[:, None], jnp.zeros((n_j, AUG - d - 1), f32)], axis=1)

    e_pad = _ceil_to(e, NW * CHUNK)
    n_chunks = e_pad // (NW * CHUNK)
    idx_i2 = jnp.pad(sparse_i_sample.astype(jnp.int32), (0, e_pad - e)
                     ).reshape(-1, SUB)
    idx_j2 = jnp.pad(sparse_j_sample.astype(jnp.int32), (0, e_pad - e)
                     ).reshape(-1, SUB)

    sc_params = pltpu.CompilerParams(use_tc_tiling_on_sc=False,
                                     needs_layout_passes=False)
    mesh = plsc.VectorSubcoreMesh(core_axis_name="c", subcore_axis_name="s",
                                  num_cores=NC, num_subcores=NS)
    link_fn = pl.kernel(
        functools.partial(_link_kernel_body, n_chunks, e),
        out_type=jax.ShapeDtypeStruct((NW, LANES), f32),
        mesh=mesh,
        compiler_params=sc_params,
        scratch_types=[
            pltpu.VMEM((NSUB, SUB), jnp.int32),
            pltpu.VMEM((NSUB, SUB), jnp.int32),
            pltpu.VMEM((CHUNK, AUG), f32),
            pltpu.VMEM((CHUNK, AUG), f32),
            pltpu.VMEM((LANES,), f32),
            pltpu.SemaphoreType.DMA,
            pltpu.SemaphoreType.DMA,
        ],
    )
    link_partials = link_fn(aug_i, aug_j, idx_i2, idx_j2)

    s_pad = _ceil_to(max(s_i, s_j), NW * 8)
    s_per_tile = s_pad // NW
    sidx_i = jnp.pad(sample_i_idx.astype(jnp.int32), (0, s_pad - s_i))
    sidx_j = jnp.pad(sample_j_idx.astype(jnp.int32), (0, s_pad - s_j))
    sample_fn = pl.kernel(
        functools.partial(_sample_kernel_body, s_per_tile),
        out_type=(jax.ShapeDtypeStruct((s_pad, AUG), f32),
                  jax.ShapeDtypeStruct((s_pad, AUG), f32)),
        mesh=plsc.VectorSubcoreMesh(core_axis_name="c", subcore_axis_name="s",
                                    num_cores=NC, num_subcores=NS),
        compiler_params=sc_params,
        scratch_types=[
            pltpu.VMEM((s_per_tile,), jnp.int32),
            pltpu.VMEM((s_per_tile, AUG), f32),
            pltpu.SemaphoreType.DMA,
        ],
    )
    rows_i_s, rows_j_s = sample_fn(aug_i, aug_j, sidx_i, sidx_j)

    exp_sum = pl.pallas_call(
        functools.partial(_dense_body, s_i, s_j),
        grid=(s_pad // S_BLK,),
        in_specs=[
            pl.BlockSpec((S_BLK, AUG), lambda i: (i, 0)),
            pl.BlockSpec((s_pad, AUG), lambda i: (0, 0)),
        ],
        out_specs=pl.BlockSpec((1, 1), lambda i: (0, 0)),
        out_shape=jax.ShapeDtypeStruct((1, 1), f32),
    )(rows_i_s, rows_j_s)

    return jnp.sum(link_partials) - exp_sum[0, 0]
